# baseline (speedup 1.0000x reference)
import functools

import jax
import jax.numpy as jnp
from jax import lax
from jax.experimental import pallas as pl
from jax.experimental.pallas import tpu as pltpu
from jax.experimental.pallas import tpu_sc as plsc

N = 512
D = 128
H = 128
E = 8192
NS_ITERS = 10

_LANES = 16
_WORKERS = 32
_ROWS_PER_W = N // 16
_WORDS_PER_W = _ROWS_PER_W * N
_EDGE_CHUNKS = E // _LANES


def _sc_counts_body(ea_hbm, eb_hbm, out_a, out_b, src_v, dst_v, acc_v):
    cid = lax.axis_index("c")
    sid = lax.axis_index("s")
    base = sid * _WORDS_PER_W
    ones = jnp.full((_LANES,), 1.0, dtype=jnp.float32)
    zeros = jnp.zeros((_LANES,), dtype=jnp.float32)

    def do_type(e_hbm, out):
        pltpu.sync_copy(e_hbm.at[0], src_v)
        pltpu.sync_copy(e_hbm.at[1], dst_v)

        def zero_body(i, carry):
            acc_v[pl.ds(i * _LANES, _LANES)] = zeros
            return carry

        lax.fori_loop(0, _WORDS_PER_W // _LANES, zero_body, 0, unroll=8)

        def edge_body(i, carry):
            s = src_v[pl.ds(i * _LANES, _LANES)]
            d = dst_v[pl.ds(i * _LANES, _LANES)]
            m = (s >= sid * _ROWS_PER_W) & (s < (sid + 1) * _ROWS_PER_W)
            idx = s * N + d - base
            idx = jnp.where(m, idx, 0)
            plsc.addupdate_scatter(acc_v, [idx], ones, mask=m)
            return carry

        lax.fori_loop(0, _EDGE_CHUNKS, edge_body, 0, unroll=8)

        pltpu.sync_copy(acc_v, out.at[pl.ds(base, _WORDS_PER_W)])

    @pl.when(cid == 0)
    def _():
        do_type(ea_hbm, out_a)

    @pl.when(cid == 1)
    def _():
        do_type(eb_hbm, out_b)


@functools.cache
def _sc_counts():
    return pl.kernel(
        _sc_counts_body,
        out_type=(
            jax.ShapeDtypeStruct((N * N,), jnp.float32),
            jax.ShapeDtypeStruct((N * N,), jnp.float32),
        ),
        mesh=plsc.VectorSubcoreMesh(core_axis_name="c", subcore_axis_name="s",
                                    num_cores=2, num_subcores=16),
        scratch_types=[
            pltpu.VMEM((E,), jnp.int32),
            pltpu.VMEM((E,), jnp.int32),
            pltpu.VMEM((_WORDS_PER_W,), jnp.float32),
        ],
        compiler_params=pltpu.CompilerParams(needs_layout_passes=False),
    )


def _dot(a, b, ta=False, tb=False):
    dims = (((0,) if ta else (1,), (1,) if tb else (0,)), ((), ()))
    return lax.dot_general(a, b, dims,
                           preferred_element_type=jnp.float32)


def _tc_body(x_ref, ca_ref, cb_ref, w1l_ref, w1r_ref, b1_ref, w2l_ref,
             w2r_ref, b2_ref, lw_ref, lb_ref,
             xnew_ref, adja_ref, adjb_ref, s1_ref):
    x = x_ref[...]
    ca = ca_ref[...]
    cb = cb_ref[...]

    def sage_layer(h, wl_ref, wr_ref, b_ref):
        out = jnp.zeros((N, H), dtype=jnp.float32)
        for t, c in ((0, ca), (1, cb)):
            cnt = jnp.sum(c, axis=0)
            agg = _dot(c, h, ta=True)
            mean = agg / jnp.maximum(cnt, 1.0)[:, None]
            out = out + _dot(h, wl_ref[t]) + _dot(mean, wr_ref[t]) \
                + b_ref[t][None, :]
        return out

    h1 = jnp.maximum(sage_layer(x, w1l_ref, w1r_ref, b1_ref), 0.0)
    z = sage_layer(h1, w2l_ref, w2r_ref, b2_ref)

    w = lw_ref[...]
    a_col = _dot(z * z, w)
    a_row = _dot(w, z * z, ta=True, tb=True)
    bmat = _dot(z * w[:, 0][None, :], z, tb=True)
    pre = a_col + a_row - 2.0 * bmat + lb_ref[0, 0]
    m = jax.nn.sigmoid(pre)

    rows = lax.broadcasted_iota(jnp.int32, (N, N), 0)
    cols = lax.broadcasted_iota(jnp.int32, (N, N), 1)
    eye_m = rows == cols
    eye = jnp.where(eye_m, 1.0, 0.0)
    sym = m + jnp.where(eye_m, m, 0.0)

    norm_a = jnp.sqrt(jnp.sum(sym * sym)) + 1e-8
    y = sym / norm_a
    zi = eye
    for it in range(NS_ITERS):
        t = 1.5 * eye - 0.5 * _dot(zi, y)
        y = _dot(y, t)
        if it + 1 < NS_ITERS:
            zi = _dot(t, zi)
    s1 = y * jnp.sqrt(norm_a)

    adj_a = jnp.minimum(ca, 1.0)
    adj_b = jnp.minimum(cb, 1.0)

    xnew_ref[...] = _dot(s1, x)
    adja_ref[...] = _dot(_dot(s1, adj_a), s1)
    adjb_ref[...] = _dot(_dot(s1, adj_b), s1)
    s1_ref[...] = s1


_tc_main = pl.pallas_call(
    _tc_body,
    out_shape=(
        jax.ShapeDtypeStruct((N, D), jnp.float32),
        jax.ShapeDtypeStruct((N, N), jnp.float32),
        jax.ShapeDtypeStruct((N, N), jnp.float32),
        jax.ShapeDtypeStruct((N, N), jnp.float32),
    ),
)


def kernel(x_note, W1l, W1r, b1, W2l, W2r, b2, lin_w, lin_b,
           edge_index_a, edge_index_b):
    ca_flat, cb_flat = _sc_counts()(edge_index_a, edge_index_b)
    ca = ca_flat.reshape(N, N)
    cb = cb_flat.reshape(N, N)
    lb = lin_b.reshape(1, 1)
    return _tc_main(x_note, ca, cb, W1l, W1r, b1, W2l, W2r, b2, lin_w, lb)


# speedup vs baseline: 1.0833x; 1.0833x over previous
import functools

import jax
import jax.numpy as jnp
from jax import lax
from jax.experimental import pallas as pl
from jax.experimental.pallas import tpu as pltpu
from jax.experimental.pallas import tpu_sc as plsc

N = 512
D = 128
H = 128
E = 8192
NS_ITERS = 10

_LANES = 16
_WORKERS = 32
_ROWS_PER_W = N // 16
_WORDS_PER_W = _ROWS_PER_W * N
_EDGE_CHUNKS = E // _LANES


def _sc_counts_body(ea_hbm, eb_hbm, out_a, out_b, src_v, dst_v, acc_v):
    cid = lax.axis_index("c")
    sid = lax.axis_index("s")
    base = sid * _WORDS_PER_W
    ones = jnp.full((_LANES,), 1.0, dtype=jnp.float32)
    zeros = jnp.zeros((_LANES,), dtype=jnp.float32)

    def do_type(e_hbm, out):
        pltpu.sync_copy(e_hbm.at[0], src_v)
        pltpu.sync_copy(e_hbm.at[1], dst_v)

        def zero_body(i, carry):
            acc_v[pl.ds(i * _LANES, _LANES)] = zeros
            return carry

        lax.fori_loop(0, _WORDS_PER_W // _LANES, zero_body, 0, unroll=8)

        def edge_body(i, carry):
            s = src_v[pl.ds(i * _LANES, _LANES)]
            d = dst_v[pl.ds(i * _LANES, _LANES)]
            m = (s >= sid * _ROWS_PER_W) & (s < (sid + 1) * _ROWS_PER_W)
            idx = s * N + d - base
            idx = jnp.where(m, idx, 0)
            plsc.addupdate_scatter(acc_v, [idx], ones, mask=m)
            return carry

        lax.fori_loop(0, _EDGE_CHUNKS, edge_body, 0, unroll=8)

        pltpu.sync_copy(acc_v, out.at[pl.ds(base, _WORDS_PER_W)])

    @pl.when(cid == 0)
    def _():
        do_type(ea_hbm, out_a)

    @pl.when(cid == 1)
    def _():
        do_type(eb_hbm, out_b)


@functools.cache
def _sc_counts():
    return pl.kernel(
        _sc_counts_body,
        out_type=(
            jax.ShapeDtypeStruct((N * N,), jnp.float32),
            jax.ShapeDtypeStruct((N * N,), jnp.float32),
        ),
        mesh=plsc.VectorSubcoreMesh(core_axis_name="c", subcore_axis_name="s",
                                    num_cores=2, num_subcores=16),
        scratch_types=[
            pltpu.VMEM((E,), jnp.int32),
            pltpu.VMEM((E,), jnp.int32),
            pltpu.VMEM((_WORDS_PER_W,), jnp.float32),
        ],
        compiler_params=pltpu.CompilerParams(needs_layout_passes=False),
    )


def _dot(a, b, ta=False, tb=False):
    dims = (((0,) if ta else (1,), (1,) if tb else (0,)), ((), ()))
    return lax.dot_general(a, b, dims,
                           preferred_element_type=jnp.float32)


def _tc_body(x_ref, ca_ref, cb_ref, w1l_ref, w1r_ref, b1_ref, w2l_ref,
             w2r_ref, b2_ref, lw_ref, lb_ref,
             xnew_ref, adja_ref, adjb_ref, s1_ref):
    x = x_ref[...]
    ca = ca_ref[...].reshape(N, N)
    cb = cb_ref[...].reshape(N, N)

    def sage_layer(h, wl_ref, wr_ref, b_ref):
        out = jnp.zeros((N, H), dtype=jnp.float32)
        for t, c in ((0, ca), (1, cb)):
            cnt = jnp.sum(c, axis=0)
            agg = _dot(c, h, ta=True)
            mean = agg / jnp.maximum(cnt, 1.0)[:, None]
            out = out + _dot(h, wl_ref[t]) + _dot(mean, wr_ref[t]) \
                + b_ref[t][None, :]
        return out

    h1 = jnp.maximum(sage_layer(x, w1l_ref, w1r_ref, b1_ref), 0.0)
    z = sage_layer(h1, w2l_ref, w2r_ref, b2_ref)

    w = lw_ref[...]
    a_col = _dot(z * z, w)
    a_row = _dot(w, z * z, ta=True, tb=True)
    bmat = _dot(z * w[:, 0][None, :], z, tb=True)
    pre = a_col + a_row - 2.0 * bmat + lb_ref[0, 0]
    m = jax.nn.sigmoid(pre)

    rows = lax.broadcasted_iota(jnp.int32, (N, N), 0)
    cols = lax.broadcasted_iota(jnp.int32, (N, N), 1)
    eye_m = rows == cols
    eye = jnp.where(eye_m, 1.0, 0.0)
    sym = m + jnp.where(eye_m, m, 0.0)

    norm_a = jnp.sqrt(jnp.sum(sym * sym)) + 1e-8
    y = sym / norm_a
    zi = eye
    for it in range(NS_ITERS):
        t = 1.5 * eye - 0.5 * _dot(zi, y)
        y = _dot(y, t)
        if it + 1 < NS_ITERS:
            zi = _dot(t, zi)
    s1 = y * jnp.sqrt(norm_a)

    adj_a = jnp.minimum(ca, 1.0)
    adj_b = jnp.minimum(cb, 1.0)

    xnew_ref[...] = _dot(s1, x)
    adja_ref[...] = _dot(_dot(s1, adj_a), s1)
    adjb_ref[...] = _dot(_dot(s1, adj_b), s1)
    s1_ref[...] = s1


_tc_main = pl.pallas_call(
    _tc_body,
    out_shape=(
        jax.ShapeDtypeStruct((N, D), jnp.float32),
        jax.ShapeDtypeStruct((N, N), jnp.float32),
        jax.ShapeDtypeStruct((N, N), jnp.float32),
        jax.ShapeDtypeStruct((N, N), jnp.float32),
    ),
)


def kernel(x_note, W1l, W1r, b1, W2l, W2r, b2, lin_w, lin_b,
           edge_index_a, edge_index_b):
    ca_flat, cb_flat = _sc_counts()(edge_index_a, edge_index_b)
    lb = lin_b.reshape(1, 1)
    return _tc_main(x_note, ca_flat, cb_flat, W1l, W1r, b1, W2l, W2r, b2,
                    lin_w, lb)
